# baseline (speedup 1.0000x reference)
import functools

import jax
import jax.numpy as jnp
from jax import lax
from jax.experimental import pallas as pl
from jax.experimental.pallas import tpu as pltpu
from jax.experimental.pallas import tpu_sc as plsc

_NUM_EMB = 1024
_DIM = 64
_N_TOK = 32768
_EPS = 1e-12

_BLK = 256
_GRID = _N_TOK // _BLK
_CC = 128
_NCC = _NUM_EMB // _CC


def _argmin_body(x_ref, c_ref, used_ref, idx_ref, used_out_ref, perp_ref, acc_ref):
    i = pl.program_id(0)
    x = x_ref[...]
    run_min = jnp.full((1, _BLK), jnp.inf, jnp.float32)
    run_arg = jnp.zeros((1, _BLK), jnp.int32)
    for j in range(_NCC):
        cj = c_ref[pl.ds(j * _CC, _CC), :]
        cn = jnp.sum(cj * cj, axis=1, keepdims=True)
        simT = lax.dot_general(
            cj, x, (((1,), (1,)), ((), ())), preferred_element_type=jnp.float32
        )
        dist = cn - 2.0 * simT
        m = jnp.min(dist, axis=0, keepdims=True)
        row_iota = lax.broadcasted_iota(jnp.int32, (_CC, _BLK), 0)
        a = jnp.min(
            jnp.where(dist == m, row_iota + j * _CC, _NUM_EMB), axis=0, keepdims=True
        )
        better = m < run_min
        run_arg = jnp.where(better, a, run_arg)
        run_min = jnp.minimum(run_min, m)
    idx_ref[0, :, :] = run_arg

    for j in range(_NCC):
        row_iota = lax.broadcasted_iota(jnp.int32, (_CC, _BLK), 0)
        eq = (row_iota + j * _CC == run_arg).astype(jnp.int32)
        cnt = jnp.sum(eq, axis=1, keepdims=True)

        @pl.when(i == 0)
        def _init():
            acc_ref[:, pl.ds(j, 1)] = cnt

        @pl.when(i > 0)
        def _accum():
            acc_ref[:, pl.ds(j, 1)] += cnt

    @pl.when(i == _GRID - 1)
    def _finish():
        counts = acc_ref[...]
        used_out_ref[...] = used_ref[...] + counts
        p = counts.astype(jnp.float32) * (1.0 / _N_TOK)
        perp = jnp.exp(-jnp.sum(p * jnp.log(p + _EPS), axis=(0, 1), keepdims=True))
        perp_ref[...] = perp


def _argmin_counts(flat, codebooks, used_t):
    return pl.pallas_call(
        _argmin_body,
        grid=(_GRID,),
        in_specs=[
            pl.BlockSpec((_BLK, _DIM), lambda i: (i, 0)),
            pl.BlockSpec((_NUM_EMB, _DIM), lambda i: (0, 0)),
            pl.BlockSpec((_CC, _NCC), lambda i: (0, 0)),
        ],
        out_specs=[
            pl.BlockSpec((1, 1, _BLK), lambda i: (i, 0, 0)),
            pl.BlockSpec((_CC, _NCC), lambda i: (0, 0)),
            pl.BlockSpec((1, 1), lambda i: (0, 0)),
        ],
        out_shape=[
            jax.ShapeDtypeStruct((_GRID, 1, _BLK), jnp.int32),
            jax.ShapeDtypeStruct((_CC, _NCC), jnp.int32),
            jax.ShapeDtypeStruct((1, 1), jnp.float32),
        ],
        scratch_shapes=[pltpu.VMEM((_CC, _NCC), jnp.int32)],
    )(flat, codebooks, used_t)


_NW = 32
_BPW = _N_TOK // _NW
_CH = 128
_NCH = _BPW // _CH


@functools.lru_cache(maxsize=1)
def _get_sc_gather():
    info = plsc.get_sparse_core_info()
    nc = info.num_cores
    assert nc * info.num_subcores == _NW

    @functools.partial(
        pl.kernel,
        mesh=plsc.VectorSubcoreMesh(core_axis_name="c", subcore_axis_name="s"),
        out_type=jax.ShapeDtypeStruct((_N_TOK, _DIM), jnp.float32),
        scratch_types=[
            pltpu.VMEM((_NCH, _CH), jnp.int32),
            pltpu.VMEM((_BPW, _DIM), jnp.float32),
            pltpu.SemaphoreType.DMA,
        ],
        compiler_params=pltpu.CompilerParams(use_tc_tiling_on_sc=False),
    )
    def _sc_gather(c_hbm, idx_hbm, out_hbm, idx_v, rows_v, sem):
        wid = lax.axis_index("s") * nc + lax.axis_index("c")
        base = wid * _BPW
        pltpu.sync_copy(idx_hbm.at[wid], idx_v)
        handles = []
        for ch in range(_NCH):
            handles.append(
                pltpu.async_copy(
                    c_hbm.at[idx_v.at[ch]],
                    rows_v.at[pl.ds(ch * _CH, _CH)],
                    sem,
                )
            )
        for h in handles:
            h.wait()
        pltpu.sync_copy(rows_v, out_hbm.at[pl.ds(base, _BPW)])

    return _sc_gather


def kernel(input_data, codebooks, codebooks_used):
    flat = input_data.reshape(-1, _DIM)
    used_t = codebooks_used.reshape(_NCC, _CC).T
    idx_blocks, used_out, perp = _argmin_counts(flat, codebooks, used_t)
    idx_grouped = idx_blocks.reshape(_NW, _NCH, _CH)
    quantized = _get_sc_gather()(codebooks, idx_grouped)
    quantized = quantized.reshape(input_data.shape[:-1] + (_DIM,))
    return (quantized, perp[0, 0], used_out.T.reshape(_NUM_EMB))


# speedup vs baseline: 1.3410x; 1.3410x over previous
import functools

import jax
import jax.numpy as jnp
from jax import lax
from jax.experimental import pallas as pl
from jax.experimental.pallas import tpu as pltpu
from jax.experimental.pallas import tpu_sc as plsc

_NUM_EMB = 1024
_DIM = 64
_N_TOK = 32768
_EPS = 1e-12

_BLK = 256
_GRID = _N_TOK // _BLK
_CC = 128
_NCC = _NUM_EMB // _CC


def _argmin_body(
    x_ref, c_ref, used_ref, idx_ref, used_out_ref, perp_ref,
    acc_ref, dist_ref, cp_ref, xp_ref,
):
    i = pl.program_id(0)

    @pl.when(i == 0)
    def _precompute():
        cp_ref[...] = jnp.zeros((_NUM_EMB, 2 * _DIM), jnp.float32)
        c = c_ref[...]
        cp_ref[:, pl.ds(0, _DIM)] = c
        cp_ref[:, pl.ds(_DIM, 1)] = jnp.sum(c * c, axis=1, keepdims=True)
        xp_ref[...] = jnp.zeros((_BLK, 2 * _DIM), jnp.float32)
        xp_ref[:, pl.ds(_DIM, 1)] = jnp.ones((_BLK, 1), jnp.float32)
        acc_ref[...] = jnp.zeros((_CC, _NCC), jnp.float32)

    xp_ref[:, pl.ds(0, _DIM)] = -2.0 * x_ref[...]
    xp = xp_ref[...]

    run_min = jnp.full((1, _BLK), jnp.inf, jnp.float32)
    for j in range(_NCC):
        cpj = cp_ref[pl.ds(j * _CC, _CC), :]
        dist = lax.dot_general(
            cpj, xp, (((1,), (1,)), ((), ())), preferred_element_type=jnp.float32
        )
        dist_ref[pl.ds(j * _CC, _CC), :] = dist
        run_min = jnp.minimum(run_min, jnp.min(dist, axis=0, keepdims=True))

    run_arg = jnp.full((1, _BLK), _NUM_EMB, jnp.int32)
    for j in range(_NCC):
        dist = dist_ref[pl.ds(j * _CC, _CC), :]
        row_iota = lax.broadcasted_iota(jnp.int32, (_CC, _BLK), 0)
        cand = jnp.where(dist == run_min, row_iota + j * _CC, _NUM_EMB)
        run_arg = jnp.minimum(run_arg, jnp.min(cand, axis=0, keepdims=True))
    idx_ref[0, :, :] = run_arg

    ones = jnp.ones((_BLK, 1), jnp.float32)
    for j in range(_NCC):
        row_iota = lax.broadcasted_iota(jnp.int32, (_CC, _BLK), 0)
        eq = jnp.where(row_iota + j * _CC == run_arg, 1.0, 0.0)
        cnt = lax.dot_general(
            eq, ones, (((1,), (0,)), ((), ())), preferred_element_type=jnp.float32
        )
        acc_ref[:, pl.ds(j, 1)] += cnt

    @pl.when(i == _GRID - 1)
    def _finish():
        counts = acc_ref[...]
        used_out_ref[...] = used_ref[...] + counts.astype(jnp.int32)
        p = counts * (1.0 / _N_TOK)
        perp = jnp.exp(-jnp.sum(p * jnp.log(p + _EPS), axis=(0, 1), keepdims=True))
        perp_ref[...] = perp


def _argmin_counts(flat, codebooks, used_t):
    return pl.pallas_call(
        _argmin_body,
        grid=(_GRID,),
        in_specs=[
            pl.BlockSpec((_BLK, _DIM), lambda i: (i, 0)),
            pl.BlockSpec((_NUM_EMB, _DIM), lambda i: (0, 0)),
            pl.BlockSpec((_CC, _NCC), lambda i: (0, 0)),
        ],
        out_specs=[
            pl.BlockSpec((1, 1, _BLK), lambda i: (i, 0, 0)),
            pl.BlockSpec((_CC, _NCC), lambda i: (0, 0)),
            pl.BlockSpec((1, 1), lambda i: (0, 0)),
        ],
        out_shape=[
            jax.ShapeDtypeStruct((_GRID, 1, _BLK), jnp.int32),
            jax.ShapeDtypeStruct((_CC, _NCC), jnp.int32),
            jax.ShapeDtypeStruct((1, 1), jnp.float32),
        ],
        scratch_shapes=[
            pltpu.VMEM((_CC, _NCC), jnp.float32),
            pltpu.VMEM((_NUM_EMB, _BLK), jnp.float32),
            pltpu.VMEM((_NUM_EMB, 2 * _DIM), jnp.float32),
            pltpu.VMEM((_BLK, 2 * _DIM), jnp.float32),
        ],
    )(flat, codebooks, used_t)


_NW = 32
_BPW = _N_TOK // _NW
_CH = 128
_NCH = _BPW // _CH


@functools.lru_cache(maxsize=1)
def _get_sc_gather():
    info = plsc.get_sparse_core_info()
    nc = info.num_cores
    assert nc * info.num_subcores == _NW

    @functools.partial(
        pl.kernel,
        mesh=plsc.VectorSubcoreMesh(core_axis_name="c", subcore_axis_name="s"),
        out_type=jax.ShapeDtypeStruct((_N_TOK, _DIM), jnp.float32),
        scratch_types=[
            pltpu.VMEM((_NCH, _CH), jnp.int32),
            pltpu.VMEM((_BPW, _DIM), jnp.float32),
            pltpu.SemaphoreType.DMA,
        ],
        compiler_params=pltpu.CompilerParams(use_tc_tiling_on_sc=False),
    )
    def _sc_gather(c_hbm, idx_hbm, out_hbm, idx_v, rows_v, sem):
        wid = lax.axis_index("s") * nc + lax.axis_index("c")
        base = wid * _BPW
        pltpu.sync_copy(idx_hbm.at[wid], idx_v)
        handles = []
        for ch in range(_NCH):
            handles.append(
                pltpu.async_copy(
                    c_hbm.at[idx_v.at[ch]],
                    rows_v.at[pl.ds(ch * _CH, _CH)],
                    sem,
                )
            )
        for h in handles:
            h.wait()
        pltpu.sync_copy(rows_v, out_hbm.at[pl.ds(base, _BPW)])

    return _sc_gather


def kernel(input_data, codebooks, codebooks_used):
    flat = input_data.reshape(-1, _DIM)
    used_t = codebooks_used.reshape(_NCC, _CC).T
    idx_blocks, used_out, perp = _argmin_counts(flat, codebooks, used_t)
    idx_grouped = idx_blocks.reshape(_NW, _NCH, _CH)
    quantized = _get_sc_gather()(codebooks, idx_grouped)
    quantized = quantized.reshape(input_data.shape[:-1] + (_DIM,))
    return (quantized, perp[0, 0], used_out.T.reshape(_NUM_EMB))
